# baseline (speedup 1.0000x reference)
import functools

import jax
import jax.numpy as jnp
from jax import lax
from jax.experimental import pallas as pl
from jax.experimental.pallas import tpu as pltpu
from jax.experimental.pallas import tpu_sc as plsc

H = 1024
W = 1024
D = 32
N = 2097152

NC = 2
NS = 16
L = 16
NW = NC * NS
PPW = N // NW
C = 512
NCHUNK = PPW // C
SLAB = 128
NSLAB = C // SLAB
GPS = SLAB // L
DB = D // 8
TPC = C // 128


def _make_sc_kernel():
    mesh = plsc.VectorSubcoreMesh(core_axis_name="c", subcore_axis_name="s")

    vbuf = lambda: pltpu.VMEM((C, D), jnp.bfloat16)
    ibuf = lambda: pltpu.VMEM((NSLAB, SLAB), jnp.int32)
    fbuf = lambda: pltpu.VMEM((C,), jnp.float32)

    @functools.partial(
        pl.kernel,
        mesh=mesh,
        out_type=jax.ShapeDtypeStruct((N * D // 128, 128), jnp.float32),
        compiler_params=pltpu.CompilerParams(
            use_tc_tiling_on_sc=False, needs_layout_passes=False),
        scratch_types=[
            fbuf(), fbuf(),
            fbuf(), fbuf(),
            ibuf(), ibuf(), ibuf(), ibuf(),
            ibuf(), ibuf(), ibuf(), ibuf(),
            fbuf(), fbuf(),
            fbuf(), fbuf(),
            vbuf(), vbuf(), vbuf(), vbuf(),
            vbuf(), vbuf(), vbuf(), vbuf(),
            pltpu.VMEM((DB * TPC * 8, 129), jnp.float32),
            pltpu.VMEM((DB * TPC * 8, 129), jnp.float32),
            pltpu.SemaphoreType.DMA,
            pltpu.SemaphoreType.DMA,
            pltpu.SemaphoreType.DMA,
            pltpu.SemaphoreType.DMA,
        ],
    )
    def theta_sc(ux_hbm, uy_hbm, tex_hbm, out_hbm,
                 uxa_v, uya_v, uxb_v, uyb_v,
                 i00a, i01a, i10a, i11a, i00b, i01b, i10b, i11b,
                 wxa, wya, wxb, wyb,
                 v00a, v01a, v10a, v11a, v00b, v01b, v10b, v11b,
                 outa_v, outb_v, sem0, sem1, semo0, semo1):
        cid = lax.axis_index("c")
        sid = lax.axis_index("s")
        wid = sid * NC + cid
        wbase = wid * PPW

        bufs = (
            ((uxa_v, uya_v), (i00a, i01a, i10a, i11a), wxa, wya,
             (v00a, v01a, v10a, v11a), sem0, outa_v, semo0),
            ((uxb_v, uyb_v), (i00b, i01b, i10b, i11b), wxb, wyb,
             (v00b, v01b, v10b, v11b), sem1, outb_v, semo1),
        )
        lane = lax.iota(jnp.int32, L)
        lane_lo = lane // 8
        lane_c8 = lane % 8

        def load_and_index(g, bset):
            (ux_v, uy_v), idxs, wx_v, wy_v, _, _, _, _ = bufs[bset]
            base = wbase + g * C
            pltpu.sync_copy(ux_hbm.at[pl.ds(base, C)], ux_v)
            pltpu.sync_copy(uy_hbm.at[pl.ds(base, C)], uy_v)

            def pha_slab(j, carry):
                def pha(ii, carry):
                    i = j * GPS + ii
                    u = ux_v[pl.ds(i * L, L)]
                    v = uy_v[pl.ds(i * L, L)]
                    ix = jnp.minimum(jnp.maximum(u * (W - 1.0), 0.0), W - 1.0)
                    iy = jnp.minimum(jnp.maximum(v * (H - 1.0), 0.0), H - 1.0)
                    x0 = jnp.minimum(ix.astype(jnp.int32), W - 2)
                    y0 = jnp.minimum(iy.astype(jnp.int32), H - 2)
                    wx = ix - x0.astype(jnp.float32)
                    wy = iy - y0.astype(jnp.float32)
                    i00 = y0 * W + x0
                    s = pl.ds(ii * L, L)
                    idxs[0][j, s] = i00
                    idxs[1][j, s] = i00 + 1
                    idxs[2][j, s] = i00 + W
                    idxs[3][j, s] = i00 + (W + 1)
                    wx_v[pl.ds(i * L, L)] = wx
                    wy_v[pl.ds(i * L, L)] = wy
                    return carry
                return lax.fori_loop(0, GPS, pha, carry, unroll=4)
            lax.fori_loop(0, NSLAB, pha_slab, 0)

        def fire(bset):
            _, idxs, _, _, vs, sem, _, _ = bufs[bset]
            for j in range(NSLAB):
                dst = pl.ds(j * SLAB, SLAB)
                for q in range(4):
                    pltpu.async_copy(
                        tex_hbm.at[idxs[q].at[j]], vs[q].at[dst], sem)

        def drain(bset):
            _, idxs, _, _, vs, sem, _, _ = bufs[bset]
            for j in range(NSLAB):
                dst = pl.ds(j * SLAB, SLAB)
                for q in range(4):
                    pltpu.make_async_copy(
                        tex_hbm.at[idxs[q].at[j]], vs[q].at[dst], sem).wait()

        def combine(g, bset):
            _, _, wx_v, wy_v, (v00, v01, v10, v11), _, out_v, _ = bufs[bset]

            def phc(i, carry):
                wxv = wx_v[pl.ds(i * L, L)]
                wyv = wy_v[pl.ds(i * L, L)]
                for p in range(L):
                    wxp = wxv[p]
                    wyp = wyv[p]
                    pt = i * L + p
                    tb8 = jnp.full((L,), (pt // 128) * 8, jnp.int32)
                    po = jnp.full((L,), pt % 128, jnp.int32)
                    s = pl.ds(0, D)
                    a0, a1 = plsc.unpack(
                        v00[pt, s], format=plsc.PackFormat.INTERLEAVED)
                    b0, b1 = plsc.unpack(
                        v01[pt, s], format=plsc.PackFormat.INTERLEAVED)
                    c0, c1 = plsc.unpack(
                        v10[pt, s], format=plsc.PackFormat.INTERLEAVED)
                    d0, d1 = plsc.unpack(
                        v11[pt, s], format=plsc.PackFormat.INTERLEAVED)
                    for h, (a, b, c, d) in enumerate(
                            ((a0, b0, c0, d0), (a1, b1, c1, d1))):
                        top = a + wxp * (b - a)
                        bot = c + wxp * (d - c)
                        res = top + wyp * (bot - top)
                        row = (lane_lo + 2 * h) * (TPC * 8) + tb8 + lane_c8
                        plsc.store_scatter(out_v, [row, po], res)
                return carry
            lax.fori_loop(0, C // L, phc, 0)

        def out_tiles(g, bset, do):
            _, _, _, _, _, _, out_v, semo = bufs[bset]
            tn0 = (wbase + g * C) // 128
            for tcq in range(DB):
                for tb in range(TPC):
                    row0 = (tcq * (N // 128) + tn0 + tb) * 8
                    src = out_v.at[pl.ds((tcq * TPC + tb) * 8, 8), pl.ds(0, 128)]
                    dst = out_hbm.at[pl.ds(row0, 8)]
                    if do:
                        pltpu.async_copy(src, dst, semo)
                    else:
                        pltpu.make_async_copy(src, dst, semo).wait()

        load_and_index(0, 0)
        fire(0)

        def outer(gg, carry):
            for b in range(2):
                g = 2 * gg + b
                nxt = g + 1
                if b == 0:
                    load_and_index(nxt, 1)
                    fire(1)
                else:
                    @pl.when(gg < NCHUNK // 2 - 1)
                    def _():
                        load_and_index(nxt, 0)
                        fire(0)
                drain(b)

                @pl.when(gg >= 1)
                def _():
                    out_tiles(g, b, do=False)
                combine(g, b)
                out_tiles(g, b, do=True)
            return carry
        lax.fori_loop(0, NCHUNK // 2, outer, 0)
        out_tiles(NCHUNK - 2, 0, do=False)
        out_tiles(NCHUNK - 1, 1, do=False)

    return theta_sc


_THETA_SC = _make_sc_kernel()


def _make_cast_kernel():
    mesh = plsc.VectorSubcoreMesh(core_axis_name="c", subcore_axis_name="s")
    CC = 1024
    RPW = H * W // NW
    NCC = RPW // CC

    @functools.partial(
        pl.kernel,
        mesh=mesh,
        out_type=jax.ShapeDtypeStruct((H * W, D), jnp.bfloat16),
        compiler_params=pltpu.CompilerParams(
            use_tc_tiling_on_sc=False, needs_layout_passes=False),
        scratch_types=[
            pltpu.VMEM((D, CC + 1), jnp.float32),
            pltpu.VMEM((D, CC + 1), jnp.float32),
            pltpu.VMEM((CC, D), jnp.bfloat16),
            pltpu.VMEM((CC, D), jnp.bfloat16),
            pltpu.SemaphoreType.DMA,
            pltpu.SemaphoreType.DMA,
            pltpu.SemaphoreType.DMA,
            pltpu.SemaphoreType.DMA,
        ],
    )
    def cast_sc(t32_hbm, tbf_hbm, f0, f1, g0, g1, si0, si1, so0, so1):
        cid = lax.axis_index("c")
        sid = lax.axis_index("s")
        rbase = (sid * NC + cid) * RPW
        fs, gs, sis, sos = (f0, f1), (g0, g1), (si0, si1), (so0, so1)
        lane = lax.iota(jnp.int32, L)
        lane_hi = lane + L

        def fire_in(g, b):
            pltpu.async_copy(
                t32_hbm.at[:, pl.ds(rbase + g * CC, CC)],
                fs[b].at[:, pl.ds(0, CC)], sis[b])

        def wait_in(b):
            pltpu.make_async_copy(
                t32_hbm.at[:, pl.ds(rbase, CC)],
                fs[b].at[:, pl.ds(0, CC)], sis[b]).wait()

        def fire_out(g, b):
            pltpu.async_copy(
                gs[b], tbf_hbm.at[pl.ds(rbase + g * CC, CC)], sos[b])

        def wait_out(b):
            pltpu.make_async_copy(
                gs[b], tbf_hbm.at[pl.ds(rbase, CC)], sos[b]).wait()

        fire_in(0, 0)

        def body(gg, carry):
            for b in range(2):
                g = 2 * gg + b
                if b == 0:
                    fire_in(g + 1, 1)
                else:
                    @pl.when(gg < NCC // 2 - 1)
                    def _():
                        fire_in(g + 1, 0)
                wait_in(b)

                @pl.when(gg >= 1)
                def _():
                    wait_out(b)

                def row(r, carry):
                    rs = jnp.full((L,), r, jnp.int32)
                    a = plsc.load_gather(fs[b], [lane, rs])
                    c = plsc.load_gather(fs[b], [lane_hi, rs])
                    gs[b][r, pl.ds(0, D)] = plsc.pack(
                        a, c, format=plsc.PackFormat.INTERLEAVED)
                    return carry
                lax.fori_loop(0, CC, row, 0, unroll=8)
                fire_out(g, b)
            return carry
        lax.fori_loop(0, NCC // 2, body, 0)
        wait_out(0)
        wait_out(1)

    return cast_sc


_CAST_SC = _make_cast_kernel()


def kernel(uv, tex):
    tex_t = _CAST_SC(tex[0].reshape(D, H * W))
    ux = uv[:, 0]
    uy = uv[:, 1]
    out = _THETA_SC(ux, uy, tex_t)
    return (out.reshape(D // 8, N // 128, 8, 128)
            .transpose(1, 3, 0, 2).reshape(N, D))


# speedup vs baseline: 1.0045x; 1.0045x over previous
import functools

import jax
import jax.numpy as jnp
from jax import lax
from jax.experimental import pallas as pl
from jax.experimental.pallas import tpu as pltpu
from jax.experimental.pallas import tpu_sc as plsc

H = 1024
W = 1024
D = 32
N = 2097152

NC = 2
NS = 16
L = 16
NW = NC * NS
PPW = N // NW
C = 512
NCHUNK = PPW // C
SLAB = 128
NSLAB = C // SLAB
GPS = SLAB // L
DB = D // 8
TPC = C // 128


def _make_sc_kernel():
    mesh = plsc.VectorSubcoreMesh(core_axis_name="c", subcore_axis_name="s")

    vbuf = lambda: pltpu.VMEM((C, D), jnp.bfloat16)
    ibuf = lambda: pltpu.VMEM((NSLAB, SLAB), jnp.int32)
    fbuf = lambda: pltpu.VMEM((C,), jnp.float32)

    @functools.partial(
        pl.kernel,
        mesh=mesh,
        out_type=jax.ShapeDtypeStruct((N * D // 128, 128), jnp.float32),
        compiler_params=pltpu.CompilerParams(
            use_tc_tiling_on_sc=False, needs_layout_passes=False),
        scratch_types=[
            fbuf(), fbuf(),
            fbuf(), fbuf(),
            ibuf(), ibuf(), ibuf(), ibuf(),
            ibuf(), ibuf(), ibuf(), ibuf(),
            fbuf(), fbuf(),
            fbuf(), fbuf(),
            vbuf(), vbuf(), vbuf(), vbuf(),
            vbuf(), vbuf(), vbuf(), vbuf(),
            pltpu.VMEM((DB * TPC * 8, 129), jnp.float32),
            pltpu.VMEM((DB * TPC * 8, 129), jnp.float32),
            pltpu.SemaphoreType.DMA,
            pltpu.SemaphoreType.DMA,
            pltpu.SemaphoreType.DMA,
            pltpu.SemaphoreType.DMA,
        ],
    )
    def theta_sc(ux_hbm, uy_hbm, tex_hbm, out_hbm,
                 uxa_v, uya_v, uxb_v, uyb_v,
                 i00a, i01a, i10a, i11a, i00b, i01b, i10b, i11b,
                 wxa, wya, wxb, wyb,
                 v00a, v01a, v10a, v11a, v00b, v01b, v10b, v11b,
                 outa_v, outb_v, sem0, sem1, semo0, semo1):
        cid = lax.axis_index("c")
        sid = lax.axis_index("s")
        wid = sid * NC + cid
        wbase = wid * PPW

        bufs = (
            ((uxa_v, uya_v), (i00a, i01a, i10a, i11a), wxa, wya,
             (v00a, v01a, v10a, v11a), sem0, outa_v, semo0),
            ((uxb_v, uyb_v), (i00b, i01b, i10b, i11b), wxb, wyb,
             (v00b, v01b, v10b, v11b), sem1, outb_v, semo1),
        )
        lane = lax.iota(jnp.int32, L)
        lane_lo = lane // 8
        lane_c8 = lane % 8

        def load_and_index(g, bset):
            (ux_v, uy_v), idxs, wx_v, wy_v, _, _, _, _ = bufs[bset]
            base = wbase + g * C
            pltpu.sync_copy(ux_hbm.at[pl.ds(base, C)], ux_v)
            pltpu.sync_copy(uy_hbm.at[pl.ds(base, C)], uy_v)

            def pha_slab(j, carry):
                def pha(ii, carry):
                    i = j * GPS + ii
                    u = ux_v[pl.ds(i * L, L)]
                    v = uy_v[pl.ds(i * L, L)]
                    ix = jnp.minimum(jnp.maximum(u * (W - 1.0), 0.0), W - 1.0)
                    iy = jnp.minimum(jnp.maximum(v * (H - 1.0), 0.0), H - 1.0)
                    x0 = jnp.minimum(ix.astype(jnp.int32), W - 2)
                    y0 = jnp.minimum(iy.astype(jnp.int32), H - 2)
                    wx = ix - x0.astype(jnp.float32)
                    wy = iy - y0.astype(jnp.float32)
                    i00 = y0 * W + x0
                    s = pl.ds(ii * L, L)
                    idxs[0][j, s] = i00
                    idxs[1][j, s] = i00 + 1
                    idxs[2][j, s] = i00 + W
                    idxs[3][j, s] = i00 + (W + 1)
                    wx_v[pl.ds(i * L, L)] = wx
                    wy_v[pl.ds(i * L, L)] = wy
                    return carry
                return lax.fori_loop(0, GPS, pha, carry)
            lax.fori_loop(0, NSLAB, pha_slab, 0)

        def fire(bset):
            _, idxs, _, _, vs, sem, _, _ = bufs[bset]
            for j in range(NSLAB):
                dst = pl.ds(j * SLAB, SLAB)
                for q in range(4):
                    pltpu.async_copy(
                        tex_hbm.at[idxs[q].at[j]], vs[q].at[dst], sem)

        def drain(bset):
            _, idxs, _, _, vs, sem, _, _ = bufs[bset]
            for j in range(NSLAB):
                dst = pl.ds(j * SLAB, SLAB)
                for q in range(4):
                    pltpu.make_async_copy(
                        tex_hbm.at[idxs[q].at[j]], vs[q].at[dst], sem).wait()

        def combine(g, bset):
            _, _, wx_v, wy_v, (v00, v01, v10, v11), _, out_v, _ = bufs[bset]

            def phc(i, carry):
                wxv = wx_v[pl.ds(i * L, L)]
                wyv = wy_v[pl.ds(i * L, L)]
                for p in range(L):
                    wxp = wxv[p]
                    wyp = wyv[p]
                    pt = i * L + p
                    tb8 = jnp.full((L,), (pt // 128) * 8, jnp.int32)
                    po = jnp.full((L,), pt % 128, jnp.int32)
                    s = pl.ds(0, D)
                    a0, a1 = plsc.unpack(
                        v00[pt, s], format=plsc.PackFormat.INTERLEAVED)
                    b0, b1 = plsc.unpack(
                        v01[pt, s], format=plsc.PackFormat.INTERLEAVED)
                    c0, c1 = plsc.unpack(
                        v10[pt, s], format=plsc.PackFormat.INTERLEAVED)
                    d0, d1 = plsc.unpack(
                        v11[pt, s], format=plsc.PackFormat.INTERLEAVED)
                    for h, (a, b, c, d) in enumerate(
                            ((a0, b0, c0, d0), (a1, b1, c1, d1))):
                        top = a + wxp * (b - a)
                        bot = c + wxp * (d - c)
                        res = top + wyp * (bot - top)
                        row = (lane_lo + 2 * h) * (TPC * 8) + tb8 + lane_c8
                        plsc.store_scatter(out_v, [row, po], res)
                return carry
            lax.fori_loop(0, C // L, phc, 0)

        def out_tiles(g, bset, do):
            _, _, _, _, _, _, out_v, semo = bufs[bset]
            tn0 = (wbase + g * C) // 128
            for tcq in range(DB):
                for tb in range(TPC):
                    row0 = (tcq * (N // 128) + tn0 + tb) * 8
                    src = out_v.at[pl.ds((tcq * TPC + tb) * 8, 8), pl.ds(0, 128)]
                    dst = out_hbm.at[pl.ds(row0, 8)]
                    if do:
                        pltpu.async_copy(src, dst, semo)
                    else:
                        pltpu.make_async_copy(src, dst, semo).wait()

        load_and_index(0, 0)
        fire(0)

        def outer(gg, carry):
            for b in range(2):
                g = 2 * gg + b
                nxt = g + 1
                if b == 0:
                    load_and_index(nxt, 1)
                    fire(1)
                else:
                    @pl.when(gg < NCHUNK // 2 - 1)
                    def _():
                        load_and_index(nxt, 0)
                        fire(0)
                drain(b)

                @pl.when(gg >= 1)
                def _():
                    out_tiles(g, b, do=False)
                combine(g, b)
                out_tiles(g, b, do=True)
            return carry
        lax.fori_loop(0, NCHUNK // 2, outer, 0)
        out_tiles(NCHUNK - 2, 0, do=False)
        out_tiles(NCHUNK - 1, 1, do=False)

    return theta_sc


_THETA_SC = _make_sc_kernel()


def _make_cast_kernel():
    mesh = plsc.VectorSubcoreMesh(core_axis_name="c", subcore_axis_name="s")
    CC = 1024
    RPW = H * W // NW
    NCC = RPW // CC

    @functools.partial(
        pl.kernel,
        mesh=mesh,
        out_type=jax.ShapeDtypeStruct((H * W, D), jnp.bfloat16),
        compiler_params=pltpu.CompilerParams(
            use_tc_tiling_on_sc=False, needs_layout_passes=False),
        scratch_types=[
            pltpu.VMEM((D, CC + 1), jnp.float32),
            pltpu.VMEM((D, CC + 1), jnp.float32),
            pltpu.VMEM((CC, D), jnp.bfloat16),
            pltpu.VMEM((CC, D), jnp.bfloat16),
            pltpu.SemaphoreType.DMA,
            pltpu.SemaphoreType.DMA,
            pltpu.SemaphoreType.DMA,
            pltpu.SemaphoreType.DMA,
        ],
    )
    def cast_sc(t32_hbm, tbf_hbm, f0, f1, g0, g1, si0, si1, so0, so1):
        cid = lax.axis_index("c")
        sid = lax.axis_index("s")
        rbase = (sid * NC + cid) * RPW
        fs, gs, sis, sos = (f0, f1), (g0, g1), (si0, si1), (so0, so1)
        lane = lax.iota(jnp.int32, L)
        lane_hi = lane + L

        def fire_in(g, b):
            pltpu.async_copy(
                t32_hbm.at[:, pl.ds(rbase + g * CC, CC)],
                fs[b].at[:, pl.ds(0, CC)], sis[b])

        def wait_in(b):
            pltpu.make_async_copy(
                t32_hbm.at[:, pl.ds(rbase, CC)],
                fs[b].at[:, pl.ds(0, CC)], sis[b]).wait()

        def fire_out(g, b):
            pltpu.async_copy(
                gs[b], tbf_hbm.at[pl.ds(rbase + g * CC, CC)], sos[b])

        def wait_out(b):
            pltpu.make_async_copy(
                gs[b], tbf_hbm.at[pl.ds(rbase, CC)], sos[b]).wait()

        fire_in(0, 0)

        def body(gg, carry):
            for b in range(2):
                g = 2 * gg + b
                if b == 0:
                    fire_in(g + 1, 1)
                else:
                    @pl.when(gg < NCC // 2 - 1)
                    def _():
                        fire_in(g + 1, 0)
                wait_in(b)

                @pl.when(gg >= 1)
                def _():
                    wait_out(b)

                def row(r, carry):
                    rs = jnp.full((L,), r, jnp.int32)
                    a = plsc.load_gather(fs[b], [lane, rs])
                    c = plsc.load_gather(fs[b], [lane_hi, rs])
                    gs[b][r, pl.ds(0, D)] = plsc.pack(
                        a, c, format=plsc.PackFormat.INTERLEAVED)
                    return carry
                lax.fori_loop(0, CC, row, 0)
                fire_out(g, b)
            return carry
        lax.fori_loop(0, NCC // 2, body, 0)
        wait_out(0)
        wait_out(1)

    return cast_sc


_CAST_SC = _make_cast_kernel()


def kernel(uv, tex):
    tex_t = _CAST_SC(tex[0].reshape(D, H * W))
    ux = uv[:, 0]
    uy = uv[:, 1]
    out = _THETA_SC(ux, uy, tex_t)
    return (out.reshape(D // 8, N // 128, 8, 128)
            .transpose(1, 3, 0, 2).reshape(N, D))
